# baseline (speedup 1.0000x reference)
import jax
import jax.numpy as jnp
from jax import lax
from jax.experimental import pallas as pl
from jax.experimental.pallas import tpu as pltpu
from jax.experimental.pallas import tpu_sc as plsc

NC = 2
NS = 16
K = 125


def _sc_aggregate(xc_list, col_sh, row_sh, zeros_blk, n_pad, h):
    NB = col_sh.shape[1]
    W = 16
    RPS = n_pad // NS
    KZ = 32
    RC = RPS // KZ

    def body(x0, x1, x2, x3, colh, rowh, zh,
             a0, a1, a2, a3,
             acc):
        core = lax.axis_index("c")
        sub = lax.axis_index("s")

        def scoped(colw0, roww0, colw1, roww1, stg0, stg1, g0, g1, i0, i1):
            inner(x0, x1, x2, x3, colh, rowh, zh, a0, a1, a2, a3, acc,
                  core, sub, ((colw0, roww0, i0), (colw1, roww1, i1)),
                  stg0, stg1, g0, g1)

        pl.run_scoped(
            scoped,
            pltpu.VMEM((W, K), jnp.int32),
            pltpu.VMEM((W, K), jnp.int32),
            pltpu.VMEM((W, K), jnp.int32),
            pltpu.VMEM((W, K), jnp.int32),
            pltpu.VMEM((K, h), jnp.float32),
            pltpu.VMEM((K, h), jnp.float32),
            pltpu.SemaphoreType.DMA,
            pltpu.SemaphoreType.DMA,
            pltpu.SemaphoreType.DMA,
            pltpu.SemaphoreType.DMA,
        )

    def inner(x0, x1, x2, x3, colh, rowh, zh, a0, a1, a2, a3, acc,
              core, sub, wbufs, stg0, stg1, g0, g1):

        NW = NB // W

        def run_round(tbl, out):
            base = sub * RPS
            pltpu.sync_copy(zh, acc.at[pl.ds(base, RPS)])
            plsc.subcore_barrier()

            def idx_load(w, q):
                colw, roww, isem = wbufs[q]
                woff = pl.multiple_of(w * W, W)
                pltpu.async_copy(colh.at[sub, pl.ds(woff, W)], colw, isem)
                pltpu.async_copy(rowh.at[sub, pl.ds(woff, W)], roww, isem)

            idx_load(0, 0)
            idx_load(1, 1)

            def wpair(p, carry0):
                for q in range(2):
                    colw, roww, isem = wbufs[q]
                    w = 2 * p + q
                    woff = pl.multiple_of(w * W, W)
                    pltpu.make_async_copy(
                        colh.at[sub, pl.ds(woff, W)], colw, isem).wait()
                    pltpu.make_async_copy(
                        rowh.at[sub, pl.ds(woff, W)], roww, isem).wait()
                    pltpu.async_copy(tbl.at[colw.at[0]], stg0, g0)
                    pltpu.async_copy(tbl.at[colw.at[1]], stg1, g1)

                    def step(i, carry):
                        for j, (stg, gs) in enumerate(((stg0, g0), (stg1, g1))):
                            b = 2 * i + j
                            pltpu.make_async_copy(tbl.at[colw.at[b]], stg,
                                                  gs).wait()
                            pltpu.sync_copy(stg, acc.at[roww.at[b]], add=True)

                            @pl.when(b + 2 < W)
                            def _issue():
                                pltpu.async_copy(tbl.at[colw.at[b + 2]], stg, gs)
                        return carry

                    lax.fori_loop(0, W // 2, step, 0)

                    @pl.when(w + 2 < NW)
                    def _prefetch():
                        idx_load(w + 2, q)
                return carry0

            lax.fori_loop(0, NW // 2, wpair, 0)
            plsc.subcore_barrier()
            pltpu.sync_copy(acc.at[pl.ds(base, RPS)], out.at[pl.ds(base, RPS)])

        @pl.when(core == 0)
        def _c0():
            run_round(x0, a0)
            run_round(x1, a1)

        @pl.when(core == 1)
        def _c1():
            run_round(x2, a2)
            run_round(x3, a3)

    mesh = plsc.VectorSubcoreMesh(core_axis_name="c", subcore_axis_name="s",
                                  num_cores=NC, num_subcores=NS)
    out_t = [jax.ShapeDtypeStruct((n_pad, h), jnp.float32)] * 4
    return pl.kernel(
        body,
        out_type=out_t,
        mesh=mesh,
        scratch_types=[
            pltpu.VMEM_SHARED((n_pad, h), jnp.float32),
        ],
    )(*xc_list, col_sh, row_sh, zeros_blk)


def _dense(xf, aggs, wsr, wsro, bias, wvr, wvro, n_nodes, h, bn):
    def body(xr, a0r, a1r, a2r, a3r, wsr_r, wsro_r, b_r, wvr_r, wvro_r, outr):
        x = xr[...]
        aggs_r = (a0r, a1r, a2r, a3r)
        for c in range(4):
            xc = x[:, c * h:(c + 1) * h]
            ac = aggs_r[c][...]
            wroot = wsro_r[...] if c == 0 else wvro_r[...]
            wrel = wsr_r[...] if c == 0 else wvr_r[...]
            r = lax.dot_general(xc, wroot, (((1,), (1,)), ((), ())),
                                preferred_element_type=jnp.float32)
            r = r + lax.dot_general(ac, wrel, (((1,), (1,)), ((), ())),
                                    preferred_element_type=jnp.float32)
            if c == 0:
                r = r + b_r[...]
            outr[:, c * h:(c + 1) * h] = r

    grid = (n_nodes // bn,)
    blk_n = pl.BlockSpec((bn, 4 * h), lambda i: (i, 0))
    blk_a = pl.BlockSpec((bn, h), lambda i: (i, 0))
    blk_w = pl.BlockSpec((h, h), lambda i: (0, 0))
    blk_b = pl.BlockSpec((1, h), lambda i: (0, 0))
    return pl.pallas_call(
        body,
        grid=grid,
        in_specs=[blk_n, blk_a, blk_a, blk_a, blk_a,
                  blk_w, blk_w, blk_b, blk_w, blk_w],
        out_specs=blk_n,
        out_shape=jax.ShapeDtypeStruct((n_nodes, 4 * h), jnp.float32),
    )(xf, *aggs, wsr, wsro, bias, wvr, wvro)


def kernel(x, edge_index, W_scalar_rel, W_scalar_root, b_scalar_root,
           W_vector_rel, W_vector_root):
    n, C, h = x.shape
    e = edge_index.shape[1]
    assert C == 4 and e % (NS * K) == 0
    n_pad = ((n + NS * 128 - 1) // (NS * 128)) * (NS * 128)

    row = edge_index[0].astype(jnp.int32)
    col = edge_index[1].astype(jnp.int32)
    NB = e // (NS * K)
    col_sh = col.reshape(NS, NB, K)
    row_sh = row.reshape(NS, NB, K)
    xc_list = [x[:, c, :] for c in range(4)]
    zeros_blk = jnp.zeros((n_pad // NS, h), jnp.float32)

    aggs = _sc_aggregate(xc_list, col_sh, row_sh, zeros_blk, n_pad, h)

    xf = x.reshape(n, C * h)
    out = _dense(xf, aggs, W_scalar_rel, W_scalar_root,
                 b_scalar_root.reshape(1, h), W_vector_rel, W_vector_root,
                 n, h, bn=400)
    return out.reshape(n, C, h)


# speedup vs baseline: 1.0119x; 1.0119x over previous
import jax
import jax.numpy as jnp
from jax import lax
from jax.experimental import pallas as pl
from jax.experimental.pallas import tpu as pltpu
from jax.experimental.pallas import tpu_sc as plsc

NC = 2
NS = 16
K = 125


def _sc_aggregate(xc_list, col_sh, row_sh, zeros_blk, n_pad, h):
    NB = col_sh.shape[1]
    W = 40
    RPS = n_pad // NS
    KZ = 32
    RC = RPS // KZ

    def body(x0, x1, x2, x3, colh, rowh, zh,
             a0, a1, a2, a3,
             acc):
        core = lax.axis_index("c")
        sub = lax.axis_index("s")

        def scoped(colw, roww, stg0, stg1, g0, g1):
            inner(x0, x1, x2, x3, colh, rowh, zh, a0, a1, a2, a3, acc,
                  core, sub, colw, roww, stg0, stg1, g0, g1)

        pl.run_scoped(
            scoped,
            pltpu.VMEM((W, K), jnp.int32),
            pltpu.VMEM((W, K), jnp.int32),
            pltpu.VMEM((K, h), jnp.float32),
            pltpu.VMEM((K, h), jnp.float32),
            pltpu.SemaphoreType.DMA,
            pltpu.SemaphoreType.DMA,
        )

    def inner(x0, x1, x2, x3, colh, rowh, zh, a0, a1, a2, a3, acc,
              core, sub, colw, roww, stg0, stg1, g0, g1):

        def run_round(tbl, out):
            base = sub * RPS
            pltpu.sync_copy(zh, acc.at[pl.ds(base, RPS)])
            plsc.subcore_barrier()

            def window(w, carry0):
                woff = pl.multiple_of(w * W, W)
                pltpu.sync_copy(colh.at[sub, pl.ds(woff, W)], colw)
                pltpu.sync_copy(rowh.at[sub, pl.ds(woff, W)], roww)
                pltpu.async_copy(tbl.at[colw.at[0]], stg0, g0)
                pltpu.async_copy(tbl.at[colw.at[1]], stg1, g1)

                def step(i, carry):
                    for j, (stg, gs) in enumerate(((stg0, g0), (stg1, g1))):
                        b = 2 * i + j
                        pltpu.make_async_copy(tbl.at[colw.at[b]], stg, gs).wait()
                        pltpu.sync_copy(stg, acc.at[roww.at[b]], add=True)

                        @pl.when(b + 2 < W)
                        def _issue():
                            pltpu.async_copy(tbl.at[colw.at[b + 2]], stg, gs)
                    return carry

                lax.fori_loop(0, W // 2, step, 0)
                return carry0

            lax.fori_loop(0, NB // W, window, 0)
            plsc.subcore_barrier()
            pltpu.sync_copy(acc.at[pl.ds(base, RPS)], out.at[pl.ds(base, RPS)])

        @pl.when(core == 0)
        def _c0():
            run_round(x0, a0)
            run_round(x1, a1)

        @pl.when(core == 1)
        def _c1():
            run_round(x2, a2)
            run_round(x3, a3)

    mesh = plsc.VectorSubcoreMesh(core_axis_name="c", subcore_axis_name="s",
                                  num_cores=NC, num_subcores=NS)
    out_t = [jax.ShapeDtypeStruct((n_pad, h), jnp.float32)] * 4
    return pl.kernel(
        body,
        out_type=out_t,
        mesh=mesh,
        scratch_types=[
            pltpu.VMEM_SHARED((n_pad, h), jnp.float32),
        ],
    )(*xc_list, col_sh, row_sh, zeros_blk)


def _dense(xf, aggs, wsr, wsro, bias, wvr, wvro, n_nodes, h, bn):
    def body(xr, a0r, a1r, a2r, a3r, wsr_r, wsro_r, b_r, wvr_r, wvro_r, outr):
        x = xr[...]
        aggs_r = (a0r, a1r, a2r, a3r)
        for c in range(4):
            xc = x[:, c * h:(c + 1) * h]
            ac = aggs_r[c][...]
            wroot = wsro_r[...] if c == 0 else wvro_r[...]
            wrel = wsr_r[...] if c == 0 else wvr_r[...]
            r = lax.dot_general(xc, wroot, (((1,), (1,)), ((), ())),
                                preferred_element_type=jnp.float32)
            r = r + lax.dot_general(ac, wrel, (((1,), (1,)), ((), ())),
                                    preferred_element_type=jnp.float32)
            if c == 0:
                r = r + b_r[...]
            outr[:, c * h:(c + 1) * h] = r

    grid = (n_nodes // bn,)
    blk_n = pl.BlockSpec((bn, 4 * h), lambda i: (i, 0))
    blk_a = pl.BlockSpec((bn, h), lambda i: (i, 0))
    blk_w = pl.BlockSpec((h, h), lambda i: (0, 0))
    blk_b = pl.BlockSpec((1, h), lambda i: (0, 0))
    return pl.pallas_call(
        body,
        grid=grid,
        in_specs=[blk_n, blk_a, blk_a, blk_a, blk_a,
                  blk_w, blk_w, blk_b, blk_w, blk_w],
        out_specs=blk_n,
        out_shape=jax.ShapeDtypeStruct((n_nodes, 4 * h), jnp.float32),
    )(xf, *aggs, wsr, wsro, bias, wvr, wvro)


def kernel(x, edge_index, W_scalar_rel, W_scalar_root, b_scalar_root,
           W_vector_rel, W_vector_root):
    n, C, h = x.shape
    e = edge_index.shape[1]
    assert C == 4 and e % (NS * K) == 0
    n_pad = ((n + NS * 128 - 1) // (NS * 128)) * (NS * 128)

    row = edge_index[0].astype(jnp.int32)
    col = edge_index[1].astype(jnp.int32)
    NB = e // (NS * K)
    col_sh = col.reshape(NS, NB, K)
    row_sh = row.reshape(NS, NB, K)
    xc_list = [x[:, c, :] for c in range(4)]
    zeros_blk = jnp.zeros((n_pad // NS, h), jnp.float32)

    aggs = _sc_aggregate(xc_list, col_sh, row_sh, zeros_blk, n_pad, h)

    xf = x.reshape(n, C * h)
    out = _dense(xf, aggs, W_scalar_rel, W_scalar_root,
                 b_scalar_root.reshape(1, h), W_vector_rel, W_vector_root,
                 n, h, bn=400)
    return out.reshape(n, C, h)


# speedup vs baseline: 1.0692x; 1.0566x over previous
import jax
import jax.numpy as jnp
from jax import lax
from jax.experimental import pallas as pl
from jax.experimental.pallas import tpu as pltpu
from jax.experimental.pallas import tpu_sc as plsc

NC = 2
NS = 16
K = 125


def _sc_aggregate(xc_list, col_sh, row_sh, zeros_blk, n_pad, h):
    NB = col_sh.shape[1]
    W = 40
    RPS = n_pad // NS
    KZ = 32
    RC = RPS // KZ

    def body(xr, colh, rowh, zh,
             a0, a1, a2, a3,
             acc):
        core = lax.axis_index("c")
        sub = lax.axis_index("s")

        def scoped(colw, roww, stg0, stg1, g0, g1):
            inner(xr, colh, rowh, zh, a0, a1, a2, a3, acc,
                  core, sub, colw, roww, stg0, stg1, g0, g1)

        pl.run_scoped(
            scoped,
            pltpu.VMEM((W, K), jnp.int32),
            pltpu.VMEM((W, K), jnp.int32),
            pltpu.VMEM((K, h), jnp.float32),
            pltpu.VMEM((K, h), jnp.float32),
            pltpu.SemaphoreType.DMA,
            pltpu.SemaphoreType.DMA,
        )

    def inner(xr, colh, rowh, zh, a0, a1, a2, a3, acc,
              core, sub, colw, roww, stg0, stg1, g0, g1):

        def run_round(tbl, out):
            base = sub * RPS
            pltpu.sync_copy(zh, acc.at[pl.ds(base, RPS)])
            plsc.subcore_barrier()

            def window(w, carry0):
                woff = pl.multiple_of(w * W, W)
                pltpu.sync_copy(colh.at[sub, pl.ds(woff, W)], colw)
                pltpu.sync_copy(rowh.at[sub, pl.ds(woff, W)], roww)
                pltpu.async_copy(tbl.at[colw.at[0]], stg0, g0)
                pltpu.async_copy(tbl.at[colw.at[1]], stg1, g1)

                def step(i, carry):
                    for j, (stg, gs) in enumerate(((stg0, g0), (stg1, g1))):
                        b = 2 * i + j
                        pltpu.make_async_copy(tbl.at[colw.at[b]], stg, gs).wait()
                        pltpu.sync_copy(stg, acc.at[roww.at[b]], add=True)

                        @pl.when(b + 2 < W)
                        def _issue():
                            pltpu.async_copy(tbl.at[colw.at[b + 2]], stg, gs)
                    return carry

                lax.fori_loop(0, W // 2, step, 0)
                return carry0

            lax.fori_loop(0, NB // W, window, 0)
            plsc.subcore_barrier()
            pltpu.sync_copy(acc.at[pl.ds(base, RPS)], out.at[pl.ds(base, RPS)])

        @pl.when(core == 0)
        def _c0():
            run_round(xr.at[:, 0, :], a0)
            run_round(xr.at[:, 1, :], a1)

        @pl.when(core == 1)
        def _c1():
            run_round(xr.at[:, 2, :], a2)
            run_round(xr.at[:, 3, :], a3)

    mesh = plsc.VectorSubcoreMesh(core_axis_name="c", subcore_axis_name="s",
                                  num_cores=NC, num_subcores=NS)
    out_t = [jax.ShapeDtypeStruct((n_pad, h), jnp.float32)] * 4
    return pl.kernel(
        body,
        out_type=out_t,
        mesh=mesh,
        scratch_types=[
            pltpu.VMEM_SHARED((n_pad, h), jnp.float32),
        ],
    )(xc_list, col_sh, row_sh, zeros_blk)


def _dense(xf, aggs, wsr, wsro, bias, wvr, wvro, n_nodes, h, bn):
    def body(xr, a0r, a1r, a2r, a3r, wsr_r, wsro_r, b_r, wvr_r, wvro_r, outr):
        x = xr[...]
        aggs_r = (a0r, a1r, a2r, a3r)
        for c in range(4):
            xc = x[:, c * h:(c + 1) * h]
            ac = aggs_r[c][...]
            wroot = wsro_r[...] if c == 0 else wvro_r[...]
            wrel = wsr_r[...] if c == 0 else wvr_r[...]
            r = lax.dot_general(xc, wroot, (((1,), (1,)), ((), ())),
                                preferred_element_type=jnp.float32)
            r = r + lax.dot_general(ac, wrel, (((1,), (1,)), ((), ())),
                                    preferred_element_type=jnp.float32)
            if c == 0:
                r = r + b_r[...]
            outr[:, c * h:(c + 1) * h] = r

    grid = (n_nodes // bn,)
    blk_n = pl.BlockSpec((bn, 4 * h), lambda i: (i, 0))
    blk_a = pl.BlockSpec((bn, h), lambda i: (i, 0))
    blk_w = pl.BlockSpec((h, h), lambda i: (0, 0))
    blk_b = pl.BlockSpec((1, h), lambda i: (0, 0))
    return pl.pallas_call(
        body,
        grid=grid,
        in_specs=[blk_n, blk_a, blk_a, blk_a, blk_a,
                  blk_w, blk_w, blk_b, blk_w, blk_w],
        out_specs=blk_n,
        out_shape=jax.ShapeDtypeStruct((n_nodes, 4 * h), jnp.float32),
    )(xf, *aggs, wsr, wsro, bias, wvr, wvro)


def kernel(x, edge_index, W_scalar_rel, W_scalar_root, b_scalar_root,
           W_vector_rel, W_vector_root):
    n, C, h = x.shape
    e = edge_index.shape[1]
    assert C == 4 and e % (NS * K) == 0
    n_pad = ((n + NS * 128 - 1) // (NS * 128)) * (NS * 128)

    row = edge_index[0].astype(jnp.int32)
    col = edge_index[1].astype(jnp.int32)
    NB = e // (NS * K)
    col_sh = col.reshape(NS, NB, K)
    row_sh = row.reshape(NS, NB, K)
    xc_list = [x[:, c, :] for c in range(4)]
    zeros_blk = jnp.zeros((n_pad // NS, h), jnp.float32)

    aggs = _sc_aggregate(x, col_sh, row_sh, zeros_blk, n_pad, h)

    xf = x.reshape(n, C * h)
    out = _dense(xf, aggs, W_scalar_rel, W_scalar_root,
                 b_scalar_root.reshape(1, h), W_vector_rel, W_vector_root,
                 n, h, bn=400)
    return out.reshape(n, C, h)
